# Initial kernel scaffold; baseline (speedup 1.0000x reference)
import jax
import jax.numpy as jnp
from jax import lax
from jax.experimental import pallas as pl
from jax.experimental.pallas import tpu as pltpu
from jax.experimental.pallas import tpu_sc as plsc

EMBEDDING_DIM = 64

NC = 2
NS = 16
NW = NC * NS

CHUNK = 128


def _gather_body(idx_hbm, table_hbm, out_hbm, idx_v, rows_v, gsem, osem):
    wid = lax.axis_index("s") * NC + lax.axis_index("c")
    n_chunks = idx_v.shape[0]
    pltpu.sync_copy(idx_hbm.at[pl.ds(wid * n_chunks, n_chunks)], idx_v)
    row_base = wid * n_chunks * CHUNK

    def chunk_step(c, _):
        pltpu.async_copy(table_hbm.at[idx_v.at[c]], rows_v, gsem).wait()
        cp = pltpu.make_async_copy(
            rows_v, out_hbm.at[pl.ds(row_base + c * CHUNK, CHUNK)], osem)
        cp.start()
        cp.wait()
        return 0

    lax.fori_loop(0, n_chunks, chunk_step, 0)


def kernel(antenna_indices, embeddings):
    batch, num_antennas = antenna_indices.shape
    total = batch * num_antennas
    assert total % (NW * CHUNK) == 0
    n_chunks = total // (NW * CHUNK)

    idx2d = antenna_indices.astype(jnp.int32).reshape(total // CHUNK, CHUNK)

    mesh = plsc.VectorSubcoreMesh(core_axis_name="c", subcore_axis_name="s")
    run = pl.kernel(
        _gather_body,
        out_type=jax.ShapeDtypeStruct((total, EMBEDDING_DIM), jnp.float32),
        mesh=mesh,
        scratch_types=[
            pltpu.VMEM((n_chunks, CHUNK), jnp.int32),
            pltpu.VMEM((CHUNK, EMBEDDING_DIM), jnp.float32),
            pltpu.SemaphoreType.DMA,
            pltpu.SemaphoreType.DMA,
        ],
    )
    out = run(idx2d, embeddings)
    return out.reshape(batch, num_antennas, EMBEDDING_DIM)


# speedup vs baseline: 3.5508x; 3.5508x over previous
import jax
import jax.numpy as jnp
from jax import lax
from jax.experimental import pallas as pl
from jax.experimental.pallas import tpu as pltpu
from jax.experimental.pallas import tpu_sc as plsc

EMBEDDING_DIM = 64

NC = 2
NS = 16
NW = NC * NS

CHUNK = 128


def _gather_body(idx_hbm, table_hbm, out_hbm, idx_v, rows_v, gsem, osem):
    wid = lax.axis_index("s") * NC + lax.axis_index("c")
    n_chunks = idx_v.shape[0]
    pltpu.sync_copy(idx_hbm.at[pl.ds(wid * n_chunks, n_chunks)], idx_v)
    row_base = wid * n_chunks * CHUNK

    def chunk_step(c, _):
        pltpu.async_copy(table_hbm.at[idx_v.at[c]], rows_v, gsem).wait()
        cp = pltpu.make_async_copy(
            rows_v, out_hbm.at[pl.ds(row_base + c * CHUNK, CHUNK)], osem)
        cp.start()
        cp.wait()
        return 0

    lax.fori_loop(0, n_chunks, chunk_step, 0)


def kernel(antenna_indices, embeddings):
    batch, num_antennas = antenna_indices.shape
    total = batch * num_antennas
    assert total % (NW * CHUNK) == 0
    n_chunks = total // (NW * CHUNK)

    idx2d = antenna_indices.astype(jnp.int32).reshape(total // CHUNK, CHUNK)

    mesh = plsc.VectorSubcoreMesh(core_axis_name="c", subcore_axis_name="s")
    run = pl.kernel(
        _gather_body,
        out_type=jax.ShapeDtypeStruct((total, EMBEDDING_DIM), jnp.float32),
        mesh=mesh,
        scratch_types=[
            pltpu.VMEM((n_chunks, CHUNK), jnp.int32),
            pltpu.VMEM((CHUNK, EMBEDDING_DIM), jnp.float32),
            pltpu.SemaphoreType.DMA,
            pltpu.SemaphoreType.DMA,
        ],
        compiler_params=pltpu.CompilerParams(use_tc_tiling_on_sc=False),
    )
    out = run(idx2d, embeddings)
    return out.reshape(batch, num_antennas, EMBEDDING_DIM)


# speedup vs baseline: 4.2808x; 1.2056x over previous
import jax
import jax.numpy as jnp
from jax import lax
from jax.experimental import pallas as pl
from jax.experimental.pallas import tpu as pltpu
from jax.experimental.pallas import tpu_sc as plsc

EMBEDDING_DIM = 64

NC = 2
NS = 16
NW = NC * NS

CHUNK = 128
NBUF = 8


def _gather_body(idx_hbm, table_hbm, out_hbm, idx_v, rows_v, gsem, osem):
    wid = lax.axis_index("s") * NC + lax.axis_index("c")
    n_chunks = idx_v.shape[0]
    pltpu.sync_copy(idx_hbm.at[pl.ds(wid * n_chunks, n_chunks)], idx_v)
    row_base = wid * n_chunks * CHUNK

    def start_gather(g, b):
        pltpu.make_async_copy(
            table_hbm.at[idx_v.at[g]], rows_v.at[b], gsem.at[b]).start()

    def wait_gather(g, b):
        pltpu.make_async_copy(
            table_hbm.at[idx_v.at[g]], rows_v.at[b], gsem.at[b]).wait()

    def out_copy(g, b):
        return pltpu.make_async_copy(
            rows_v.at[b], out_hbm.at[pl.ds(row_base + g * CHUNK, CHUNK)],
            osem.at[b])

    for b in range(NBUF):
        start_gather(b, b)

    @pl.loop(0, n_chunks - NBUF, step=NBUF)
    def _(i):
        for b in range(NBUF):
            g = i + b
            wait_gather(g, b)
            out_copy(g, b).start()
            out_copy(g, b).wait()
            start_gather(g + NBUF, b)

    for b in range(NBUF):
        g = n_chunks - NBUF + b
        wait_gather(g, b)
        out_copy(g, b).start()
    for b in range(NBUF):
        g = n_chunks - NBUF + b
        out_copy(g, b).wait()


def kernel(antenna_indices, embeddings):
    batch, num_antennas = antenna_indices.shape
    total = batch * num_antennas
    assert total % (NW * CHUNK) == 0
    n_chunks = total // (NW * CHUNK)

    idx2d = antenna_indices.astype(jnp.int32).reshape(total // CHUNK, CHUNK)

    mesh = plsc.VectorSubcoreMesh(core_axis_name="c", subcore_axis_name="s")
    run = pl.kernel(
        _gather_body,
        out_type=jax.ShapeDtypeStruct((total, EMBEDDING_DIM), jnp.float32),
        mesh=mesh,
        scratch_types=[
            pltpu.VMEM((n_chunks, CHUNK), jnp.int32),
            pltpu.VMEM((NBUF, CHUNK, EMBEDDING_DIM), jnp.float32),
            pltpu.SemaphoreType.DMA((NBUF,)),
            pltpu.SemaphoreType.DMA((NBUF,)),
        ],
        compiler_params=pltpu.CompilerParams(use_tc_tiling_on_sc=False),
    )
    out = run(idx2d, embeddings)
    return out.reshape(batch, num_antennas, EMBEDDING_DIM)
